# baseline (speedup 1.0000x reference)
import functools
import math

import jax
import jax.numpy as jnp
from jax import lax
from jax.experimental import pallas as pl
from jax.experimental.pallas import tpu as pltpu
from jax.experimental.pallas import tpu_sc as plsc

NC = 2
NS = 16
L = 16
NW = NC * NS


def _sc_embed(tok_r, tab, n_chunks, C, V, D, scale):
    N = NW * n_chunks * C
    mesh = plsc.VectorSubcoreMesh(core_axis_name="c", subcore_axis_name="s")

    @functools.partial(
        pl.kernel,
        out_type=jax.ShapeDtypeStruct((N, D), jnp.float32),
        mesh=mesh,
        scratch_types=[
            pltpu.VMEM((V, D), jnp.float32),
            pltpu.HBM((V, D), jnp.float32),
            pltpu.VMEM((n_chunks, C), jnp.int32),
            pltpu.VMEM((2, C, D), jnp.float32),
            pltpu.SemaphoreType.DMA,
            pltpu.SemaphoreType.DMA,
            pltpu.SemaphoreType.DMA,
            pltpu.SemaphoreType.DMA,
        ],
    )
    def k(tok_hbm, tab_hbm, out_hbm, tab_v, tab_sh, idx_v, rows_v,
          g0, g1, o0, o1):
        cid = lax.axis_index("c")
        sid = lax.axis_index("s")
        wid = sid * NC + cid
        base = wid * n_chunks * C

        @pl.when(sid == 0)
        def _():
            pltpu.sync_copy(tab_hbm, tab_v)

            def scale_body(i, carry):
                for r in range(V):
                    sl = pl.ds(i * L, L)
                    tab_v[r, sl] = tab_v[r, sl] * scale
                return carry

            lax.fori_loop(0, D // L, scale_body, 0)
            pltpu.sync_copy(tab_v, tab_sh)

        plsc.subcore_barrier()

        pltpu.sync_copy(tok_hbm.at[wid], idx_v)

        gsem = (g0, g1)
        osem = (o0, o1)

        def gather(j, b):
            pltpu.async_copy(tab_sh.at[idx_v.at[j]], rows_v.at[b], gsem[b])

        def gather_wait(j, b):
            pltpu.make_async_copy(
                tab_sh.at[idx_v.at[j]], rows_v.at[b], gsem[b]).wait()

        def out_start(j, b):
            pltpu.async_copy(
                rows_v.at[b], out_hbm.at[pl.ds(base + j * C, C)], osem[b])

        def out_wait(j, b):
            pltpu.make_async_copy(
                rows_v.at[b], out_hbm.at[pl.ds(base + j * C, C)], osem[b]).wait()

        gather(0, 0)

        def body(j0):
            for b in range(2):
                j = j0 + b
                gather_wait(j, b)
                out_start(j, b)

                @pl.when(j + 1 < n_chunks)
                def _():
                    @pl.when(j > 0)
                    def _():
                        out_wait(j - 1, 1 - b)

                    gather(j + 1, 1 - b)

        pl.loop(0, n_chunks, step=2)(body)
        out_wait(n_chunks - 1, (n_chunks - 1) % 2)

    return k(tok_r, tab)


def kernel(tokens, emb_table):
    B, S = tokens.shape
    V, D = emb_table.shape
    N = B * S
    scale = math.sqrt(D)

    C = 32
    n_chunks = N // (NW * C)
    assert N == NW * n_chunks * C

    tok_r = tokens.reshape(NW, n_chunks, C).astype(jnp.int32)
    out = _sc_embed(tok_r, emb_table, n_chunks, C, V, D, scale)
    return out.reshape(B, S, D)


# speedup vs baseline: 1.9576x; 1.9576x over previous
import functools
import math

import jax
import jax.numpy as jnp
from jax import lax
from jax.experimental import pallas as pl
from jax.experimental.pallas import tpu as pltpu
from jax.experimental.pallas import tpu_sc as plsc

NC = 2
NS = 16
L = 16
NW = NC * NS
NB = 4


def _sc_embed(tok_r, tab, n_chunks, C, V, D, scale):
    N = NW * n_chunks * C
    mesh = plsc.VectorSubcoreMesh(core_axis_name="c", subcore_axis_name="s")

    @functools.partial(
        pl.kernel,
        out_type=jax.ShapeDtypeStruct((N, D), jnp.float32),
        mesh=mesh,
        scratch_types=[
            pltpu.HBM((8 * V, D), jnp.float32),
            pltpu.VMEM((n_chunks, C), jnp.int32),
            pltpu.VMEM((NB * C, D), jnp.float32),
        ] + [pltpu.SemaphoreType.DMA] * (2 * NB),
    )
    def k(tok_hbm, tab_hbm, out_hbm, tab_all, idx_v, rows_v,
          *sems):
        gsem = sems[:NB]
        osem = sems[NB:]
        cid = lax.axis_index("c")
        sid = lax.axis_index("s")
        wid = sid * NC + cid
        base = wid * n_chunks * C

        @pl.when(sid == 0)
        def _():
            stg = rows_v.at[pl.ds(0, V)]
            pltpu.sync_copy(tab_hbm, stg)

            def scale_body(i, carry):
                for r in range(V):
                    sl = pl.ds(i * L, L)
                    rows_v[r, sl] = rows_v[r, sl] * scale
                return carry

            lax.fori_loop(0, D // L, scale_body, 0)
            for q in range(4):
                pltpu.sync_copy(stg, tab_all.at[pl.ds((cid * 4 + q) * V, V)])

        plsc.subcore_barrier()

        pltpu.sync_copy(tok_hbm.at[wid], idx_v)
        copy_row0 = (cid * 4 + sid % 4) * V

        def add_off(j, carry):
            for cc in range(C // L):
                sl = pl.ds(cc * L, L)
                idx_v[j, sl] = idx_v[j, sl] + copy_row0
            return carry

        lax.fori_loop(0, n_chunks, add_off, 0)

        def gather(j, b):
            pltpu.async_copy(
                tab_all.at[idx_v.at[j]], rows_v.at[pl.ds(b * C, C)], gsem[b])

        def gather_wait(j, b):
            pltpu.make_async_copy(
                tab_all.at[idx_v.at[j]], rows_v.at[pl.ds(b * C, C)], gsem[b]).wait()

        def out_start(j, b):
            pltpu.async_copy(
                rows_v.at[pl.ds(b * C, C)], out_hbm.at[pl.ds(base + j * C, C)], osem[b])

        def out_wait(j, b):
            pltpu.make_async_copy(
                rows_v.at[pl.ds(b * C, C)], out_hbm.at[pl.ds(base + j * C, C)],
                osem[b]).wait()

        for b in range(NB - 1):
            gather(b, b)

        def body(j0):
            for b in range(NB):
                j = j0 + b
                pb = (b - 1) % NB
                gather_wait(j, b)
                out_start(j, b)

                @pl.when(j + NB - 1 < n_chunks)
                def _():
                    @pl.when(j > 0)
                    def _():
                        out_wait(j - 1, pb)

                    gather(j + NB - 1, pb)

        pl.loop(0, n_chunks, step=NB)(body)
        for i in range(NB):
            j = n_chunks - NB + i
            out_wait(j, j % NB)

    return k(tok_r, tab)


def kernel(tokens, emb_table):
    B, S = tokens.shape
    V, D = emb_table.shape
    N = B * S
    scale = math.sqrt(D)

    C = 16
    n_chunks = N // (NW * C)
    assert N == NW * n_chunks * C
    assert n_chunks % NB == 0 and C % 8 == 0

    Vp = (V + 7) // 8 * 8
    tab_p = jnp.pad(emb_table, ((0, Vp - V), (0, 0)))
    tok_r = tokens.reshape(NW, n_chunks, C).astype(jnp.int32)
    out = _sc_embed(tok_r, tab_p, n_chunks, C, Vp, D, scale)
    return out.reshape(B, S, D)


# speedup vs baseline: 2.3075x; 1.1788x over previous
import functools
import math

import jax
import jax.numpy as jnp
from jax import lax
from jax.experimental import pallas as pl
from jax.experimental.pallas import tpu as pltpu
from jax.experimental.pallas import tpu_sc as plsc

NC = 2
NS = 16
L = 16
NW = NC * NS
NB = 2


def _sc_embed(tok_r, tab, n_chunks, C, V, D, scale):
    N = NW * n_chunks * C
    mesh = plsc.VectorSubcoreMesh(core_axis_name="c", subcore_axis_name="s")

    @functools.partial(
        pl.kernel,
        out_type=jax.ShapeDtypeStruct((N, D), jnp.float32),
        mesh=mesh,
        scratch_types=[
            pltpu.HBM((16 * V, D), jnp.float32),
            pltpu.VMEM((n_chunks, C), jnp.int32),
            pltpu.VMEM((NB * C, D), jnp.float32),
        ] + [pltpu.SemaphoreType.DMA] * (2 * NB),
    )
    def k(tok_hbm, tab_hbm, out_hbm, tab_all, idx_v, rows_v,
          *sems):
        gsem = sems[:NB]
        osem = sems[NB:]
        cid = lax.axis_index("c")
        sid = lax.axis_index("s")
        wid = sid * NC + cid
        base = wid * n_chunks * C

        @pl.when(sid == 0)
        def _():
            stg = rows_v.at[pl.ds(0, V)]
            pltpu.sync_copy(tab_hbm, stg)

            def scale_body(i, carry):
                for r in range(V):
                    sl = pl.ds(i * L, L)
                    rows_v[r, sl] = rows_v[r, sl] * scale
                return carry

            lax.fori_loop(0, D // L, scale_body, 0)
            for q in range(8):
                pltpu.sync_copy(stg, tab_all.at[pl.ds((cid * 8 + q) * V, V)])

        plsc.subcore_barrier()

        pltpu.sync_copy(tok_hbm.at[wid], idx_v)
        copy_row0 = (cid * 8 + sid % 8) * V

        def add_off(j, carry):
            for cc in range(C // L):
                sl = pl.ds(cc * L, L)
                idx_v[j, sl] = idx_v[j, sl] + copy_row0
            return carry

        lax.fori_loop(0, n_chunks, add_off, 0)

        def gather(j, b):
            pltpu.async_copy(
                tab_all.at[idx_v.at[j]], rows_v.at[pl.ds(b * C, C)], gsem[b])

        def gather_wait(j, b):
            pltpu.make_async_copy(
                tab_all.at[idx_v.at[j]], rows_v.at[pl.ds(b * C, C)], gsem[b]).wait()

        def out_start(j, b):
            pltpu.async_copy(
                rows_v.at[pl.ds(b * C, C)], out_hbm.at[pl.ds(base + j * C, C)], osem[b])

        def out_wait(j, b):
            pltpu.make_async_copy(
                rows_v.at[pl.ds(b * C, C)], out_hbm.at[pl.ds(base + j * C, C)],
                osem[b]).wait()

        for b in range(NB - 1):
            gather(b, b)

        def body(j0):
            for b in range(NB):
                j = j0 + b
                pb = (b - 1) % NB
                gather_wait(j, b)
                out_start(j, b)

                @pl.when(j + NB - 1 < n_chunks)
                def _():
                    @pl.when(j > 0)
                    def _():
                        out_wait(j - 1, pb)

                    gather(j + NB - 1, pb)

        pl.loop(0, n_chunks, step=NB)(body)
        for i in range(NB):
            j = n_chunks - NB + i
            out_wait(j, j % NB)

    return k(tok_r, tab)


def kernel(tokens, emb_table):
    B, S = tokens.shape
    V, D = emb_table.shape
    N = B * S
    scale = math.sqrt(D)

    C = 16
    n_chunks = N // (NW * C)
    assert N == NW * n_chunks * C
    assert n_chunks % NB == 0 and C % 8 == 0

    Vp = (V + 7) // 8 * 8
    tab_p = jnp.pad(emb_table, ((0, Vp - V), (0, 0)))
    tok_r = tokens.reshape(NW, n_chunks, C).astype(jnp.int32)
    out = _sc_embed(tok_r, tab_p, n_chunks, C, Vp, D, scale)
    return out.reshape(B, S, D)


# speedup vs baseline: 5.1661x; 2.2388x over previous
import functools
import math

import jax
import jax.numpy as jnp
from jax import lax
from jax.experimental import pallas as pl
from jax.experimental.pallas import tpu as pltpu
from jax.experimental.pallas import tpu_sc as plsc

NC = 2
NS = 16
L = 16
NW = NC * NS
R = 16
MAXQ = 4


def _sc_embed(tok_r, tab, n_per_w, V, Vp, D, scale):
    N = NW * n_per_w
    NSLOT = n_per_w // R + V
    mesh = plsc.VectorSubcoreMesh(core_axis_name="c", subcore_axis_name="s")

    @functools.partial(
        pl.kernel,
        out_type=jax.ShapeDtypeStruct((N, D), jnp.float32),
        mesh=mesh,
        scratch_types=[
            pltpu.VMEM((n_per_w,), jnp.int32),
            pltpu.VMEM((Vp, D), jnp.float32),
            pltpu.VMEM((2 * R, D), jnp.float32),
            pltpu.VMEM((NSLOT, R), jnp.int32),
            pltpu.VMEM((1, D), jnp.float32),
            pltpu.SMEM((Vp,), jnp.int32),
            pltpu.SemaphoreType.DMA,
        ],
    )
    def k(tok_hbm, tab_hbm, out_hbm, idx_v, tabv, rep, pos2d, rowfix, cur,
          ssem):
        cid = lax.axis_index("c")
        sid = lax.axis_index("s")
        wid = sid * NC + cid
        base = wid * n_per_w

        pltpu.sync_copy(tok_hbm.at[wid], idx_v)
        pltpu.sync_copy(tab_hbm, tabv)

        def scale_body(i, carry):
            for r in range(Vp):
                sl = pl.ds(i * L, L)
                tabv[r, sl] = tabv[r, sl] * scale
            return carry

        lax.fori_loop(0, D // L, scale_body, 0)

        iota = lax.iota(jnp.int32, L)
        lane0 = iota == 0

        for v in range(Vp):
            cur[v] = jnp.int32(0)

        def hist(g, carry):
            tv = idx_v[pl.ds(g * L, L)]
            for l in range(L):
                t = tv[l]
                cur[t] = cur[t] + 1
            return carry

        lax.fori_loop(0, n_per_w // L, hist, 0)

        counts = [cur[v] for v in range(V)]
        slots = [(counts[v] + (R - 1)) // R for v in range(V)]
        slotbase = []
        acc = jnp.int32(0)
        for v in range(V):
            slotbase.append(acc)
            acc = acc + slots[v]

        trash = jnp.broadcast_to(base, (L,)).astype(jnp.int32)

        def fill(s, carry):
            pos2d[s] = trash
            return carry

        lax.fori_loop(0, NSLOT, fill, 0)

        for v in range(V):
            cur[v] = slotbase[v] * R

        def place(g, carry):
            tv = idx_v[pl.ds(g * L, L)]
            for l in range(L):
                t = tv[l]
                d = cur[t]
                dr = d // R
                m = iota == d % R
                row = pos2d[dr]
                val = jnp.broadcast_to(base + g * L + l, (L,)).astype(jnp.int32)
                pos2d[dr] = jnp.where(m, val, row)
                cur[t] = d + 1
            return carry

        lax.fori_loop(0, n_per_w // L, place, 0)

        def wait_one():
            pltpu.make_async_copy(
                rep.at[pl.ds(0, R)], out_hbm.at[pos2d.at[0]], ssem).wait()

        issued_t = jnp.int32(0)
        waited_t = jnp.int32(0)
        scur = jnp.int32(0)
        done_after = []

        for v in range(V):
            p = (v % 2) * R

            if v >= 2:
                def dr(i, carry):
                    wait_one()
                    return carry

                lax.fori_loop(waited_t, done_after[v - 2], dr, 0)
                waited_t = jnp.maximum(waited_t, done_after[v - 2])

            def bld(c, carry, _p=p, _v=v):
                sl = pl.ds(c * L, L)
                val = tabv[_v, sl]
                for r in range(R):
                    rep[_p + r, sl] = val
                return carry

            lax.fori_loop(0, D // L, bld, 0)

            def iss(i, carry, _p=p):
                issd, wtd = carry
                full = issd - wtd >= MAXQ

                @pl.when(full)
                def _():
                    wait_one()

                pltpu.async_copy(
                    rep.at[pl.ds(_p, R)], out_hbm.at[pos2d.at[scur + i]], ssem)
                return issd + 1, jnp.where(full, wtd + 1, wtd)

            issued_t, waited_t = lax.fori_loop(
                0, slots[v], iss, (issued_t, waited_t))
            scur = scur + slots[v]
            done_after.append(issued_t)

        def dr_all(i, carry):
            wait_one()
            return carry

        lax.fori_loop(waited_t, issued_t, dr_all, 0)

        tok0 = idx_v[pl.ds(0, L)][0]

        def fix(c, carry):
            sl = pl.ds(c * L, L)
            val = jnp.zeros((L,), jnp.float32)
            for v in range(V):
                val = val + tabv[v, sl] * (tok0 == v).astype(jnp.float32)
            rowfix[0, sl] = val
            return carry

        lax.fori_loop(0, D // L, fix, 0)
        pltpu.sync_copy(rowfix, out_hbm.at[pl.ds(base, 1)])

    return k(tok_r, tab)


def kernel(tokens, emb_table):
    B, S = tokens.shape
    V, D = emb_table.shape
    N = B * S
    scale = math.sqrt(D)

    n_per_w = N // NW
    assert N == NW * n_per_w and n_per_w % R == 0

    Vp = (V + 7) // 8 * 8
    tab_p = jnp.pad(emb_table, ((0, Vp - V), (0, 0)))
    tok_r = tokens.reshape(NW, n_per_w).astype(jnp.int32)
    out = _sc_embed(tok_r, tab_p, n_per_w, V, Vp, D, scale)
    return out.reshape(B, S, D)


# speedup vs baseline: 5.2653x; 1.0192x over previous
import functools
import math

import jax
import jax.numpy as jnp
from jax import lax
from jax.experimental import pallas as pl
from jax.experimental.pallas import tpu as pltpu
from jax.experimental.pallas import tpu_sc as plsc

NC = 2
NS = 16
L = 16
NW = NC * NS
R = 16
MAXQ = 6


def _sc_embed(tok_r, tab, n_per_w, V, Vp, D, scale):
    N = NW * n_per_w
    NSLOT = n_per_w // R + V
    mesh = plsc.VectorSubcoreMesh(core_axis_name="c", subcore_axis_name="s")

    @functools.partial(
        pl.kernel,
        out_type=jax.ShapeDtypeStruct((N, D), jnp.float32),
        mesh=mesh,
        scratch_types=[
            pltpu.VMEM((n_per_w,), jnp.int32),
            pltpu.VMEM((Vp, D), jnp.float32),
            pltpu.VMEM((2 * R, D), jnp.float32),
            pltpu.VMEM((NSLOT, R), jnp.int32),
            pltpu.VMEM((1, D), jnp.float32),
            pltpu.SMEM((Vp,), jnp.int32),
            pltpu.SemaphoreType.DMA,
        ],
    )
    def k(tok_hbm, tab_hbm, out_hbm, idx_v, tabv, rep, pos2d, rowfix, cur,
          ssem):
        cid = lax.axis_index("c")
        sid = lax.axis_index("s")
        wid = sid * NC + cid
        base = wid * n_per_w

        pltpu.sync_copy(tok_hbm.at[wid], idx_v)
        pltpu.sync_copy(tab_hbm, tabv)

        def scale_body(i, carry):
            for r in range(Vp):
                sl = pl.ds(i * L, L)
                tabv[r, sl] = tabv[r, sl] * scale
            return carry

        lax.fori_loop(0, D // L, scale_body, 0)

        iota = lax.iota(jnp.int32, L)
        lane0 = iota == 0

        def hist(g, accs):
            tv = idx_v[pl.ds(g * L, L)]
            one = jnp.ones((L,), jnp.int32)
            zero = jnp.zeros((L,), jnp.int32)
            return tuple(
                accs[v] + jnp.where(tv == v, one, zero) for v in range(V))

        accs = lax.fori_loop(
            0, n_per_w // L, hist,
            tuple(jnp.zeros((L,), jnp.int32) for _ in range(V)))
        counts = []
        for v in range(V):
            c = accs[v][0]
            for l in range(1, L):
                c = c + accs[v][l]
            counts.append(c)
        slots = [(counts[v] + (R - 1)) // R for v in range(V)]
        slotbase = []
        acc = jnp.int32(0)
        for v in range(V):
            slotbase.append(acc)
            acc = acc + slots[v]

        trash = jnp.broadcast_to(base, (L,)).astype(jnp.int32)

        def fill(s, carry):
            pos2d[s] = trash
            return carry

        lax.fori_loop(0, NSLOT, fill, 0)

        for v in range(V):
            cur[v] = slotbase[v] * R

        def place(g, carry):
            tv = idx_v[pl.ds(g * L, L)]
            for l in range(L):
                t = tv[l]
                d = cur[t]
                dr = d // R
                m = iota == d % R
                row = pos2d[dr]
                val = jnp.broadcast_to(base + g * L + l, (L,)).astype(jnp.int32)
                pos2d[dr] = jnp.where(m, val, row)
                cur[t] = d + 1
            return carry

        lax.fori_loop(0, n_per_w // L, place, 0)

        def wait_one():
            pltpu.make_async_copy(
                rep.at[pl.ds(0, R)], out_hbm.at[pos2d.at[0]], ssem).wait()

        issued_t = jnp.int32(0)
        waited_t = jnp.int32(0)
        scur = jnp.int32(0)
        done_after = []

        for v in range(V):
            p = (v % 2) * R

            if v >= 2:
                def dr(i, carry):
                    wait_one()
                    return carry

                lax.fori_loop(waited_t, done_after[v - 2], dr, 0)
                waited_t = jnp.maximum(waited_t, done_after[v - 2])

            def bld(c, carry, _p=p, _v=v):
                sl = pl.ds(c * L, L)
                val = tabv[_v, sl]
                for r in range(R):
                    rep[_p + r, sl] = val
                return carry

            @pl.when(slots[v] > 0)
            def _():
                lax.fori_loop(0, D // L, bld, 0)

            def iss(i, carry, _p=p):
                issd, wtd = carry
                full = issd - wtd >= MAXQ

                @pl.when(full)
                def _():
                    wait_one()

                pltpu.async_copy(
                    rep.at[pl.ds(_p, R)], out_hbm.at[pos2d.at[scur + i]], ssem)
                return issd + 1, jnp.where(full, wtd + 1, wtd)

            issued_t, waited_t = lax.fori_loop(
                0, slots[v], iss, (issued_t, waited_t))
            scur = scur + slots[v]
            done_after.append(issued_t)

        def dr_all(i, carry):
            wait_one()
            return carry

        lax.fori_loop(waited_t, issued_t, dr_all, 0)

        tok0 = idx_v[pl.ds(0, L)][0]

        def fix(c, carry):
            sl = pl.ds(c * L, L)
            val = jnp.zeros((L,), jnp.float32)
            for v in range(V):
                val = val + tabv[v, sl] * (tok0 == v).astype(jnp.float32)
            rowfix[0, sl] = val
            return carry

        lax.fori_loop(0, D // L, fix, 0)
        pltpu.sync_copy(rowfix, out_hbm.at[pl.ds(base, 1)])

    return k(tok_r, tab)


def kernel(tokens, emb_table):
    B, S = tokens.shape
    V, D = emb_table.shape
    N = B * S
    scale = math.sqrt(D)

    n_per_w = N // NW
    assert N == NW * n_per_w and n_per_w % R == 0

    Vp = (V + 7) // 8 * 8
    tab_p = jnp.pad(emb_table, ((0, Vp - V), (0, 0)))
    tok_r = tokens.reshape(NW, n_per_w).astype(jnp.int32)
    out = _sc_embed(tok_r, tab_p, n_per_w, V, Vp, D, scale)
    return out.reshape(B, S, D)


# speedup vs baseline: 5.3312x; 1.0125x over previous
import functools
import math

import jax
import jax.numpy as jnp
from jax import lax
from jax.experimental import pallas as pl
from jax.experimental.pallas import tpu as pltpu
from jax.experimental.pallas import tpu_sc as plsc

NC = 2
NS = 16
L = 16
NW = NC * NS
R = 16
MAXQ = 8


def _sc_embed(tok_r, tab, n_per_w, V, Vp, D, scale):
    N = NW * n_per_w
    NSLOT = n_per_w // R + V
    mesh = plsc.VectorSubcoreMesh(core_axis_name="c", subcore_axis_name="s")

    @functools.partial(
        pl.kernel,
        out_type=jax.ShapeDtypeStruct((N, D), jnp.float32),
        mesh=mesh,
        scratch_types=[
            pltpu.VMEM((n_per_w,), jnp.int32),
            pltpu.VMEM((Vp, D), jnp.float32),
            pltpu.VMEM((2 * R, D), jnp.float32),
            pltpu.VMEM((NSLOT, R), jnp.int32),
            pltpu.VMEM((1, D), jnp.float32),
            pltpu.SMEM((Vp,), jnp.int32),
            pltpu.SemaphoreType.DMA,
            pltpu.SemaphoreType.DMA,
        ],
    )
    def k(tok_hbm, tab_hbm, out_hbm, idx_v, tabv, rep, pos2d, rowfix, cur,
          ssem, tsem):
        cid = lax.axis_index("c")
        sid = lax.axis_index("s")
        wid = sid * NC + cid
        base = wid * n_per_w

        pltpu.async_copy(tab_hbm, tabv, tsem)
        pltpu.sync_copy(tok_hbm.at[wid], idx_v)

        iota = lax.iota(jnp.int32, L)
        lane0 = iota == 0

        def hist(g, accs):
            tv = idx_v[pl.ds(g * L, L)]
            one = jnp.ones((L,), jnp.int32)
            zero = jnp.zeros((L,), jnp.int32)
            return tuple(
                accs[v] + jnp.where(tv == v, one, zero) for v in range(V))

        accs = lax.fori_loop(
            0, n_per_w // L, hist,
            tuple(jnp.zeros((L,), jnp.int32) for _ in range(V)))
        counts = []
        for v in range(V):
            c = accs[v][0]
            for l in range(1, L):
                c = c + accs[v][l]
            counts.append(c)
        slots = [(counts[v] + (R - 1)) // R for v in range(V)]
        slotbase = []
        acc = jnp.int32(0)
        for v in range(V):
            slotbase.append(acc)
            acc = acc + slots[v]

        trash = jnp.broadcast_to(base, (L,)).astype(jnp.int32)

        def fill(s, carry):
            pos2d[s] = trash
            return carry

        lax.fori_loop(0, NSLOT, fill, 0)

        for v in range(V):
            cur[v] = slotbase[v] * R

        def place(g, carry):
            tv = idx_v[pl.ds(g * L, L)]
            for l in range(L):
                t = tv[l]
                d = cur[t]
                dr = d >> 4
                m = iota == (d & (R - 1))
                row = pos2d[dr]
                val = jnp.broadcast_to(base + g * L + l, (L,)).astype(jnp.int32)
                pos2d[dr] = jnp.where(m, val, row)
                cur[t] = d + 1
            return carry

        lax.fori_loop(0, n_per_w // L, place, 0)

        pltpu.make_async_copy(tab_hbm, tabv, tsem).wait()

        def scale_body(i, carry):
            for r in range(Vp):
                sl = pl.ds(i * L, L)
                tabv[r, sl] = tabv[r, sl] * scale
            return carry

        lax.fori_loop(0, D // L, scale_body, 0)

        def wait_one():
            pltpu.make_async_copy(
                rep.at[pl.ds(0, R)], out_hbm.at[pos2d.at[0]], ssem).wait()

        issued_t = jnp.int32(0)
        waited_t = jnp.int32(0)
        scur = jnp.int32(0)
        done_after = []

        for v in range(V):
            p = (v % 2) * R

            if v >= 2:
                def dr(i, carry):
                    wait_one()
                    return carry

                lax.fori_loop(waited_t, done_after[v - 2], dr, 0)
                waited_t = jnp.maximum(waited_t, done_after[v - 2])

            def bld(c, carry, _p=p, _v=v):
                sl = pl.ds(c * L, L)
                val = tabv[_v, sl]
                for r in range(R):
                    rep[_p + r, sl] = val
                return carry

            @pl.when(slots[v] > 0)
            def _():
                lax.fori_loop(0, D // L, bld, 0)

            def iss(i, carry, _p=p):
                issd, wtd = carry
                full = issd - wtd >= MAXQ

                @pl.when(full)
                def _():
                    wait_one()

                pltpu.async_copy(
                    rep.at[pl.ds(_p, R)], out_hbm.at[pos2d.at[scur + i]], ssem)
                return issd + 1, jnp.where(full, wtd + 1, wtd)

            issued_t, waited_t = lax.fori_loop(
                0, slots[v], iss, (issued_t, waited_t))
            scur = scur + slots[v]
            done_after.append(issued_t)

        def dr_all(i, carry):
            wait_one()
            return carry

        lax.fori_loop(waited_t, issued_t, dr_all, 0)

        tok0 = idx_v[pl.ds(0, L)][0]

        def fix(c, carry):
            sl = pl.ds(c * L, L)
            val = jnp.zeros((L,), jnp.float32)
            for v in range(V):
                val = val + tabv[v, sl] * (tok0 == v).astype(jnp.float32)
            rowfix[0, sl] = val
            return carry

        lax.fori_loop(0, D // L, fix, 0)
        pltpu.sync_copy(rowfix, out_hbm.at[pl.ds(base, 1)])

    return k(tok_r, tab)


def kernel(tokens, emb_table):
    B, S = tokens.shape
    V, D = emb_table.shape
    N = B * S
    scale = math.sqrt(D)

    n_per_w = N // NW
    assert N == NW * n_per_w and n_per_w % R == 0

    Vp = (V + 7) // 8 * 8
    tab_p = jnp.pad(emb_table, ((0, Vp - V), (0, 0)))
    tok_r = tokens.reshape(NW, n_per_w).astype(jnp.int32)
    out = _sc_embed(tok_r, tab_p, n_per_w, V, Vp, D, scale)
    return out.reshape(B, S, D)


# speedup vs baseline: 5.3325x; 1.0002x over previous
import functools
import math

import jax
import jax.numpy as jnp
from jax import lax
from jax.experimental import pallas as pl
from jax.experimental.pallas import tpu as pltpu
from jax.experimental.pallas import tpu_sc as plsc

NC = 2
NS = 16
L = 16
NW = NC * NS
R = 16
MAXQ = 12


def _sc_embed(tok_r, tab, n_per_w, V, Vp, D, scale):
    N = NW * n_per_w
    NSLOT = n_per_w // R + V
    mesh = plsc.VectorSubcoreMesh(core_axis_name="c", subcore_axis_name="s")

    @functools.partial(
        pl.kernel,
        out_type=jax.ShapeDtypeStruct((N, D), jnp.float32),
        mesh=mesh,
        scratch_types=[
            pltpu.VMEM((n_per_w,), jnp.int32),
            pltpu.VMEM((Vp, D), jnp.float32),
            pltpu.VMEM((2 * R, D), jnp.float32),
            pltpu.VMEM((NSLOT, R), jnp.int32),
            pltpu.VMEM((1, D), jnp.float32),
            pltpu.SMEM((Vp,), jnp.int32),
            pltpu.SemaphoreType.DMA,
            pltpu.SemaphoreType.DMA,
        ],
    )
    def k(tok_hbm, tab_hbm, out_hbm, idx_v, tabv, rep, pos2d, rowfix, cur,
          ssem, tsem):
        cid = lax.axis_index("c")
        sid = lax.axis_index("s")
        wid = sid * NC + cid
        base = wid * n_per_w

        pltpu.async_copy(tab_hbm, tabv, tsem)
        pltpu.sync_copy(tok_hbm.at[wid], idx_v)

        iota = lax.iota(jnp.int32, L)
        lane0 = iota == 0

        def hist(g, accs):
            tv = idx_v[pl.ds(g * L, L)]
            one = jnp.ones((L,), jnp.int32)
            zero = jnp.zeros((L,), jnp.int32)
            return tuple(
                accs[v] + jnp.where(tv == v, one, zero) for v in range(V))

        accs = lax.fori_loop(
            0, n_per_w // L, hist,
            tuple(jnp.zeros((L,), jnp.int32) for _ in range(V)))
        counts = []
        for v in range(V):
            c = accs[v][0]
            for l in range(1, L):
                c = c + accs[v][l]
            counts.append(c)
        slots = [(counts[v] + (R - 1)) // R for v in range(V)]
        slotbase = []
        acc = jnp.int32(0)
        for v in range(V):
            slotbase.append(acc)
            acc = acc + slots[v]

        trash = jnp.broadcast_to(base, (L,)).astype(jnp.int32)

        def fill(s, carry):
            pos2d[s] = trash
            return carry

        lax.fori_loop(0, NSLOT, fill, 0)

        for v in range(V):
            cur[v] = slotbase[v] * R

        def place(g, carry):
            tv = idx_v[pl.ds(g * L, L)]
            for l in range(L):
                t = tv[l]
                d = cur[t]
                dr = d >> 4
                m = iota == (d & (R - 1))
                row = pos2d[dr]
                val = jnp.broadcast_to(base + g * L + l, (L,)).astype(jnp.int32)
                pos2d[dr] = jnp.where(m, val, row)
                cur[t] = d + 1
            return carry

        lax.fori_loop(0, n_per_w // L, place, 0)

        pltpu.make_async_copy(tab_hbm, tabv, tsem).wait()

        def scale_body(i, carry):
            for r in range(Vp):
                sl = pl.ds(i * L, L)
                tabv[r, sl] = tabv[r, sl] * scale
            return carry

        lax.fori_loop(0, D // L, scale_body, 0)

        def wait_one():
            pltpu.make_async_copy(
                rep.at[pl.ds(0, R)], out_hbm.at[pos2d.at[0]], ssem).wait()

        issued_t = jnp.int32(0)
        waited_t = jnp.int32(0)
        scur = jnp.int32(0)
        done_after = []

        for v in range(V):
            p = (v % 2) * R

            if v >= 2:
                def dr(i, carry):
                    wait_one()
                    return carry

                lax.fori_loop(waited_t, done_after[v - 2], dr, 0)
                waited_t = jnp.maximum(waited_t, done_after[v - 2])

            def bld(c, carry, _p=p, _v=v):
                sl = pl.ds(c * L, L)
                val = tabv[_v, sl]
                for r in range(R):
                    rep[_p + r, sl] = val
                return carry

            @pl.when(slots[v] > 0)
            def _():
                lax.fori_loop(0, D // L, bld, 0)

            def iss(i, carry, _p=p):
                issd, wtd = carry
                full = issd - wtd >= MAXQ

                @pl.when(full)
                def _():
                    wait_one()

                pltpu.async_copy(
                    rep.at[pl.ds(_p, R)], out_hbm.at[pos2d.at[scur + i]], ssem)
                return issd + 1, jnp.where(full, wtd + 1, wtd)

            issued_t, waited_t = lax.fori_loop(
                0, slots[v], iss, (issued_t, waited_t))
            scur = scur + slots[v]
            done_after.append(issued_t)

        def dr_all(i, carry):
            wait_one()
            return carry

        lax.fori_loop(waited_t, issued_t, dr_all, 0)

        tok0 = idx_v[pl.ds(0, L)][0]

        def fix(c, carry):
            sl = pl.ds(c * L, L)
            val = jnp.zeros((L,), jnp.float32)
            for v in range(V):
                val = val + tabv[v, sl] * (tok0 == v).astype(jnp.float32)
            rowfix[0, sl] = val
            return carry

        lax.fori_loop(0, D // L, fix, 0)
        pltpu.sync_copy(rowfix, out_hbm.at[pl.ds(base, 1)])

    return k(tok_r, tab)


def kernel(tokens, emb_table):
    B, S = tokens.shape
    V, D = emb_table.shape
    N = B * S
    scale = math.sqrt(D)

    n_per_w = N // NW
    assert N == NW * n_per_w and n_per_w % R == 0

    Vp = (V + 7) // 8 * 8
    tab_p = jnp.pad(emb_table, ((0, Vp - V), (0, 0)))
    tok_r = tokens.reshape(NW, n_per_w).astype(jnp.int32)
    out = _sc_embed(tok_r, tab_p, n_per_w, V, Vp, D, scale)
    return out.reshape(B, S, D)


# speedup vs baseline: 5.3402x; 1.0014x over previous
import functools
import math

import jax
import jax.numpy as jnp
from jax import lax
from jax.experimental import pallas as pl
from jax.experimental.pallas import tpu as pltpu
from jax.experimental.pallas import tpu_sc as plsc

NC = 2
NS = 16
L = 16
NW = NC * NS
R = 16
MAXQ = 8


def _sc_embed(tok_r, tab, n_per_w, V, Vp, D, scale):
    N = NW * n_per_w
    NSLOT = n_per_w // R + V
    mesh = plsc.VectorSubcoreMesh(core_axis_name="c", subcore_axis_name="s")

    @functools.partial(
        pl.kernel,
        out_type=jax.ShapeDtypeStruct((N, D), jnp.float32),
        mesh=mesh,
        scratch_types=[
            pltpu.VMEM((n_per_w,), jnp.int32),
            pltpu.VMEM((Vp, D), jnp.float32),
            pltpu.VMEM((2 * R, D), jnp.float32),
            pltpu.VMEM((NSLOT, R), jnp.int32),
            pltpu.VMEM((1, D), jnp.float32),
            pltpu.SMEM((Vp,), jnp.int32),
            pltpu.SemaphoreType.DMA,
            pltpu.SemaphoreType.DMA,
        ],
    )
    def k(tok_hbm, tab_hbm, out_hbm, idx_v, tabv, rep, pos2d, rowfix, cur,
          ssem, tsem):
        cid = lax.axis_index("c")
        sid = lax.axis_index("s")
        wid = sid * NC + cid
        base = wid * n_per_w

        pltpu.async_copy(tab_hbm, tabv, tsem)
        pltpu.sync_copy(tok_hbm.at[wid], idx_v)

        iota = lax.iota(jnp.int32, L)

        def hist(g, accs):
            tv = idx_v[pl.ds(g * L, L)]
            one = jnp.ones((L,), jnp.int32)
            zero = jnp.zeros((L,), jnp.int32)
            return tuple(
                accs[v] + jnp.where(tv == v, one, zero) for v in range(V))

        accs = lax.fori_loop(
            0, n_per_w // L, hist,
            tuple(jnp.zeros((L,), jnp.int32) for _ in range(V)))
        counts = []
        for v in range(V):
            c = accs[v][0]
            for l in range(1, L):
                c = c + accs[v][l]
            counts.append(c)
        slots = [(counts[v] + (R - 1)) // R for v in range(V)]
        slotbase = []
        acc = jnp.int32(0)
        for v in range(V):
            slotbase.append(acc)
            acc = acc + slots[v]

        trash = jnp.broadcast_to(base, (L,)).astype(jnp.int32)

        def fill(s, carry):
            pos2d[s] = trash
            return carry

        lax.fori_loop(0, NSLOT, fill, 0)

        for v in range(V):
            cur[v] = slotbase[v] * R

        def place(g, carry):
            tv = idx_v[pl.ds(g * L, L)]
            for l in range(L):
                t = tv[l]
                d = cur[t]
                dr = d >> 4
                m = iota == (d & (R - 1))
                row = pos2d[dr]
                val = jnp.broadcast_to(base + g * L + l, (L,)).astype(jnp.int32)
                pos2d[dr] = jnp.where(m, val, row)
                cur[t] = d + 1
            return carry

        lax.fori_loop(0, n_per_w // L, place, 0)

        pltpu.make_async_copy(tab_hbm, tabv, tsem).wait()

        def scale_body(i, carry):
            for r in range(Vp):
                sl = pl.ds(i * L, L)
                tabv[r, sl] = tabv[r, sl] * scale
            return carry

        lax.fori_loop(0, D // L, scale_body, 0)

        def wait_one():
            pltpu.make_async_copy(
                rep.at[pl.ds(0, R)], out_hbm.at[pos2d.at[0]], ssem).wait()

        issued_t = jnp.int32(0)
        waited_t = jnp.int32(0)
        scur = jnp.int32(0)
        done_after = []

        for v in range(V):
            p = (v % 2) * R

            if v >= 2:
                def dr(i, carry):
                    wait_one()
                    return carry

                lax.fori_loop(waited_t, done_after[v - 2], dr, 0)
                waited_t = jnp.maximum(waited_t, done_after[v - 2])

            def bld(c, carry, _p=p, _v=v):
                sl = pl.ds(c * L, L)
                val = tabv[_v, sl]
                for r in range(R):
                    rep[_p + r, sl] = val
                return carry

            @pl.when(slots[v] > 0)
            def _():
                lax.fori_loop(0, D // L, bld, 0)

            def iss(i, carry, _p=p):
                issd, wtd = carry
                full = issd - wtd >= MAXQ

                @pl.when(full)
                def _():
                    wait_one()

                pltpu.async_copy(
                    rep.at[pl.ds(_p, R)], out_hbm.at[pos2d.at[scur + i]], ssem)
                return issd + 1, jnp.where(full, wtd + 1, wtd)

            issued_t, waited_t = lax.fori_loop(
                0, slots[v], iss, (issued_t, waited_t))
            scur = scur + slots[v]
            done_after.append(issued_t)

        def dr_all(i, carry):
            wait_one()
            return carry

        lax.fori_loop(waited_t, issued_t, dr_all, 0)

        tok0 = idx_v[pl.ds(0, L)][0]

        def fix(c, carry):
            sl = pl.ds(c * L, L)
            val = jnp.zeros((L,), jnp.float32)
            for v in range(V):
                val = val + tabv[v, sl] * (tok0 == v).astype(jnp.float32)
            rowfix[0, sl] = val
            return carry

        lax.fori_loop(0, D // L, fix, 0)
        pltpu.sync_copy(rowfix, out_hbm.at[pl.ds(base, 1)])

    return k(tok_r, tab)


def kernel(tokens, emb_table):
    B, S = tokens.shape
    V, D = emb_table.shape
    N = B * S
    scale = math.sqrt(D)

    n_per_w = N // NW
    assert N == NW * n_per_w and n_per_w % R == 0

    Vp = (V + 7) // 8 * 8
    tab_p = jnp.pad(emb_table, ((0, Vp - V), (0, 0)))
    tok_r = tokens.reshape(NW, n_per_w).astype(jnp.int32)
    out = _sc_embed(tok_r, tab_p, n_per_w, V, Vp, D, scale)
    return out.reshape(B, S, D)
